# baseline (speedup 1.0000x reference)
import functools

import jax
import jax.numpy as jnp
from jax import lax
from jax.experimental import pallas as pl
from jax.experimental.pallas import tpu as pltpu
from jax.experimental.pallas import tpu_sc as plsc

N = 10000
E = 320000
F = 128
H = 128
C = 40

NCORE = 2
NSUB = 16
LANES = 16

DUMMY = N
N_T = 10112
ROWS_PER_TILE = N_T // NSUB
N_D = 10240
DEG_PER_TILE = N_D // NSUB
E_PAD = 327680
EDGES_PER_TILE32 = E_PAD // 32
CHUNK = 128

R_BLK = 2528
GRID_R = N_T // R_BLK

_mesh = plsc.VectorSubcoreMesh(
    core_axis_name="c", subcore_axis_name="s", num_cores=NCORE,
    num_subcores=NSUB)

f32 = jnp.float32


E_REAL_PER_TILE = E // 32
E_TAIL = EDGES_PER_TILE32 - E_REAL_PER_TILE


@functools.partial(
    pl.kernel,
    mesh=_mesh,
    compiler_params=pltpu.CompilerParams(needs_layout_passes=False),
    out_type=(
        jax.ShapeDtypeStruct((N_D,), f32),
        jax.ShapeDtypeStruct((N_D,), f32),
        jax.ShapeDtypeStruct((N_D,), f32),
        jax.ShapeDtypeStruct((N_D,), f32),
        jax.ShapeDtypeStruct((E_PAD,), jnp.int32),
        jax.ShapeDtypeStruct((E_PAD,), jnp.int32),
    ),
    scratch_types=dict(
        esrc=pltpu.VMEM((E_REAL_PER_TILE,), jnp.int32),
        edst=pltpu.VMEM((E_REAL_PER_TILE,), jnp.int32),
        tail=pltpu.VMEM((E_TAIL,), jnp.int32),
        hsrc=pltpu.VMEM((N_D,), f32),
        hdst=pltpu.VMEM((N_D,), f32),
        red=pltpu.VMEM((NSUB, 2, DEG_PER_TILE), f32),
        res=pltpu.VMEM((2, DEG_PER_TILE), f32),
        stage=pltpu.VMEM_SHARED((NSUB, 2, N_D), f32),
    ),
)
def _sc_degrees(src_e, dst_e, do0, do1, di0, di1, spad, dpad,
                esrc, edst, tail, hsrc, hdst, red, res, stage):
  cid = lax.axis_index("c")
  sid = lax.axis_index("s")
  wid = sid * NCORE + cid

  zeros16 = jnp.zeros((LANES,), f32)

  def zero_hist(i, _):
    hsrc[pl.ds(i * LANES, LANES)] = zeros16
    hdst[pl.ds(i * LANES, LANES)] = zeros16
    return 0
  lax.fori_loop(0, N_D // LANES, zero_hist, 0)

  base = wid * E_REAL_PER_TILE
  pltpu.sync_copy(src_e.at[pl.ds(base, E_REAL_PER_TILE)], esrc)
  pltpu.sync_copy(dst_e.at[pl.ds(base, E_REAL_PER_TILE)], edst)

  dummy16 = jnp.full((LANES,), DUMMY, jnp.int32)
  for i in range(E_TAIL // LANES):
    tail[pl.ds(i * LANES, LANES)] = dummy16
  obase = wid * EDGES_PER_TILE32
  pltpu.sync_copy(esrc, spad.at[pl.ds(obase, E_REAL_PER_TILE)])
  pltpu.sync_copy(edst, dpad.at[pl.ds(obase, E_REAL_PER_TILE)])
  pltpu.sync_copy(tail, spad.at[pl.ds(obase + E_REAL_PER_TILE, E_TAIL)])
  pltpu.sync_copy(tail, dpad.at[pl.ds(obase + E_REAL_PER_TILE, E_TAIL)])

  ones16 = jnp.ones((LANES,), f32)

  def scatter_edges(i, _):
    off = i * LANES
    plsc.addupdate_scatter(hsrc, [esrc[pl.ds(off, LANES)]], ones16)
    plsc.addupdate_scatter(hdst, [edst[pl.ds(off, LANES)]], ones16)
    return 0
  lax.fori_loop(0, E_REAL_PER_TILE // LANES, scatter_edges, 0)

  pltpu.sync_copy(hsrc, stage.at[sid, 0])
  pltpu.sync_copy(hdst, stage.at[sid, 1])
  plsc.subcore_barrier()

  nbase = sid * DEG_PER_TILE
  pltpu.sync_copy(stage.at[:, :, pl.ds(nbase, DEG_PER_TILE)], red)

  def reduce_chunk(n, _):
    off = n * LANES
    for w in range(2):
      acc = red[0, w, pl.ds(off, LANES)]
      for t in range(1, NSUB):
        acc = acc + red[t, w, pl.ds(off, LANES)]
      res[w, pl.ds(off, LANES)] = acc
    return 0
  lax.fori_loop(0, DEG_PER_TILE // LANES, reduce_chunk, 0)

  rows = pl.ds(nbase, DEG_PER_TILE)

  @pl.when(cid == 0)
  def _():
    pltpu.sync_copy(res.at[0], do0.at[rows])
    pltpu.sync_copy(res.at[1], di0.at[rows])

  @pl.when(cid == 1)
  def _():
    pltpu.sync_copy(res.at[0], do1.at[rows])
    pltpu.sync_copy(res.at[1], di1.at[rows])


def _make_sc_agg(D, split32, NBUF, LA):
  ntile = 32 if split32 else 16
  nchunk = E_PAD // (ntile * CHUNK)
  NG = nchunk // NBUF
  ZROWS = ROWS_PER_TILE // 8

  @functools.partial(
      pl.kernel,
      mesh=_mesh,
      compiler_params=pltpu.CompilerParams(use_tc_tiling_on_sc=False),
      out_type=jax.ShapeDtypeStruct((N_T, 128), f32),
      scratch_types=dict(
          es=pltpu.VMEM((3, NBUF, CHUNK), jnp.int32),
          ed=pltpu.VMEM((3, NBUF, CHUNK), jnp.int32),
          gbuf=pltpu.VMEM((NBUF, CHUNK, D), f32),
          zbuf=pltpu.VMEM((ZROWS, D), f32),
          gsem=pltpu.SemaphoreType.DMA((NBUF,)),
          ssem=pltpu.SemaphoreType.DMA((NBUF,)),
          lsem=pltpu.SemaphoreType.DMA((3,)),
          acc=pltpu.VMEM_SHARED((N_T, D), f32),
          tab=pltpu.VMEM_SHARED((N_T, D), f32),
      ),
  )
  def agg(xs, src3, dst3, pout, es, ed, gbuf, zbuf, gsem, ssem,
          lsem, acc, tab):
    cid = lax.axis_index("c")
    sid = lax.axis_index("s")
    wid = sid * NCORE + cid if split32 else sid

    myrows = pl.ds(sid * ROWS_PER_TILE, ROWS_PER_TILE)
    scol = pl.ds(0, D) if split32 else pl.ds(cid * D, D)
    pltpu.sync_copy(xs.at[myrows, scol], tab.at[myrows])

    zeros16 = jnp.zeros((LANES,), f32)

    def zero_rows(j, _):
      for k in range(D // LANES):
        zbuf[j, pl.ds(k * LANES, LANES)] = zeros16
      return 0
    lax.fori_loop(0, ZROWS, zero_rows, 0)
    for z in range(ROWS_PER_TILE // ZROWS):
      pltpu.sync_copy(
          zbuf, acc.at[pl.ds(sid * ROWS_PER_TILE + z * ZROWS, ZROWS)])

    def start_load(g, slot):
      base = g * NBUF
      pltpu.async_copy(src3.at[wid, pl.ds(base, NBUF)], es.at[slot],
                       lsem.at[slot])
      pltpu.async_copy(dst3.at[wid, pl.ds(base, NBUF)], ed.at[slot],
                       lsem.at[slot])

    def wait_load(slot):
      pltpu.make_async_copy(src3.at[wid, pl.ds(0, NBUF)], es.at[slot],
                            lsem.at[slot]).wait()
      pltpu.make_async_copy(dst3.at[wid, pl.ds(0, NBUF)], ed.at[slot],
                            lsem.at[slot]).wait()

    def start_gather(idxref, slot):
      pltpu.async_copy(tab.at[idxref], gbuf.at[slot], gsem.at[slot])

    def wait_gather(slot):
      pltpu.make_async_copy(
          tab.at[es.at[0, 0]], gbuf.at[slot], gsem.at[slot]).wait()

    def wait_scatter(slot):
      pltpu.make_async_copy(
          gbuf.at[slot], acc.at[ed.at[0, 0]], ssem.at[slot]).wait()

    start_load(0, 0)
    wait_load(0)
    start_load(1, 1)
    plsc.subcore_barrier()
    for b in range(LA):
      start_gather(es.at[0, b], b)

    def group_body(g, _):
      cur = lax.rem(g, 3)
      nxt = lax.rem(g + 1, 3)

      @pl.when(g + 1 < NG)
      def _():
        wait_load(nxt)

      for b in range(NBUF):
        j = g * NBUF + b
        jn = j + LA
        slot_n = (b + LA) % NBUF

        @pl.when(jn < nchunk)
        def _():
          @pl.when(j >= LA)
          def _():
            wait_scatter(slot_n)
          if b < NBUF - LA:
            start_gather(es.at[cur, b + LA], slot_n)
          else:
            start_gather(es.at[nxt, b + LA - NBUF], slot_n)

        wait_gather(b)
        pltpu.async_copy(gbuf.at[b], acc.at[ed.at[cur, b]], ssem.at[b],
                         add=True)

      @pl.when(g + 2 < NG)
      def _():
        start_load(g + 2, lax.rem(g + 2, 3))
      return 0
    lax.fori_loop(0, NG, group_body, 0)

    for b in range(NBUF):
      wait_scatter(b)

    plsc.subcore_barrier()

    rows = pl.ds(sid * ROWS_PER_TILE, ROWS_PER_TILE)
    ocol = pl.ds(cid * D, D)
    pltpu.sync_copy(acc.at[rows], pout.at[rows, ocol])

  return agg


_sc_agg64 = _make_sc_agg(64, split32=False, NBUF=4, LA=2)
_sc_agg48 = _make_sc_agg(48, split32=True, NBUF=8, LA=4)


def _norm(d):
  return lax.rsqrt(jnp.where(d > 0.0, d, 1.0))


def _prescale_body(nf, do_c, di_c, xs, nsb, ndb):
  ns = _norm(do_c[...])
  nd = _norm(di_c[...])
  x = nf[...]
  xs[...] = x * ns
  nsb[...] = jnp.broadcast_to(ns, x.shape)
  ndb[...] = jnp.broadcast_to(nd, x.shape)


def _layer_body(a1, ndb, nsb, w1, b1, w2, out):
  h = jnp.dot(a1[...] * ndb[...], w1[...], preferred_element_type=f32)
  h = jax.nn.relu(h + b1[...])
  out[...] = jnp.dot(h, w2[...], preferred_element_type=f32) * nsb[...]


def _final_body(pp, ndb, b2, out):
  p = pp[...]
  out[...] = (p[:, :C] + p[:, 48:48 + C]) * ndb[:, :C] + b2[...]


def _row_spec(width):
  return pl.BlockSpec((R_BLK, width), lambda r: (r, 0))


def _full_spec(shape):
  return pl.BlockSpec(shape, lambda r: tuple(0 for _ in shape))


_tc_prescale = pl.pallas_call(
    _prescale_body,
    grid=(GRID_R,),
    in_specs=[_row_spec(128), _row_spec(1), _row_spec(1)],
    out_specs=[_row_spec(128)] * 3,
    out_shape=[jax.ShapeDtypeStruct((N_T, 128), f32)] * 3,
)

_tc_layer = pl.pallas_call(
    _layer_body,
    grid=(GRID_R,),
    in_specs=[_row_spec(128), _row_spec(128), _row_spec(128),
              _full_spec((128, 128)), _full_spec((1, 128)),
              _full_spec((128, 128))],
    out_specs=[_row_spec(128)],
    out_shape=[jax.ShapeDtypeStruct((N_T, 128), f32)],
)

F_BLK = 2000
_tc_final = pl.pallas_call(
    _final_body,
    grid=(N // F_BLK,),
    in_specs=[pl.BlockSpec((F_BLK, 128), lambda r: (r, 0)),
              pl.BlockSpec((F_BLK, 128), lambda r: (r, 0)),
              _full_spec((1, C))],
    out_specs=[pl.BlockSpec((F_BLK, C), lambda r: (r, 0))],
    out_shape=[jax.ShapeDtypeStruct((N, C), f32)],
)


def kernel(n_feats, edge_index, W1, b1, W2, b2):
  do0, do1, di0, di1, srcp, dstp = _sc_degrees(
      edge_index[0], edge_index[1])
  src16 = srcp.reshape(16, E_PAD // (16 * CHUNK), CHUNK)
  dst16 = dstp.reshape(16, E_PAD // (16 * CHUNK), CHUNK)
  src32 = srcp.reshape(32, E_PAD // (32 * CHUNK), CHUNK)
  dst32 = dstp.reshape(32, E_PAD // (32 * CHUNK), CHUNK)
  do_c = (do0 + do1).reshape(N_D, 1)
  di_c = (di0 + di1).reshape(N_D, 1)

  xs, nsb, ndb = _tc_prescale(n_feats, do_c, di_c)
  a1 = _sc_agg64(xs, src16, dst16)

  w2p = jnp.pad(W2, ((0, 0), (0, 128 - C)))
  (t2,) = _tc_layer(a1, ndb, nsb, W1, b1.reshape(1, H), w2p)

  pp = _sc_agg48(t2, src32, dst32)
  (outp,) = _tc_final(pp, ndb, b2.reshape(1, C))
  return outp


# speedup vs baseline: 1.0404x; 1.0404x over previous
import functools

import jax
import jax.numpy as jnp
from jax import lax
from jax.experimental import pallas as pl
from jax.experimental.pallas import tpu as pltpu
from jax.experimental.pallas import tpu_sc as plsc

N = 10000
E = 320000
F = 128
H = 128
C = 40

NCORE = 2
NSUB = 16
LANES = 16

DUMMY = N
N_T = 10112
ROWS_PER_TILE = N_T // NSUB
N_D = 10240
DEG_PER_TILE = N_D // NSUB
E_PAD = 327680
EDGES_PER_TILE32 = E_PAD // 32
CHUNK = 128

R_BLK = 2528
GRID_R = N_T // R_BLK

_mesh = plsc.VectorSubcoreMesh(
    core_axis_name="c", subcore_axis_name="s", num_cores=NCORE,
    num_subcores=NSUB)

f32 = jnp.float32


E_REAL_PER_TILE = E // 32
E_TAIL = EDGES_PER_TILE32 - E_REAL_PER_TILE


@functools.partial(
    pl.kernel,
    mesh=_mesh,
    compiler_params=pltpu.CompilerParams(
        needs_layout_passes=False, use_tc_tiling_on_sc=False),
    out_type=(
        jax.ShapeDtypeStruct((N_D,), f32),
        jax.ShapeDtypeStruct((N_D,), f32),
        jax.ShapeDtypeStruct((N_D,), f32),
        jax.ShapeDtypeStruct((N_D,), f32),
        jax.ShapeDtypeStruct((E_PAD,), jnp.int32),
        jax.ShapeDtypeStruct((E_PAD,), jnp.int32),
    ),
    scratch_types=dict(
        esrc=pltpu.VMEM((E_REAL_PER_TILE,), jnp.int32),
        edst=pltpu.VMEM((E_REAL_PER_TILE,), jnp.int32),
        tail=pltpu.VMEM((E_TAIL,), jnp.int32),
        hsrc=pltpu.VMEM((N_D,), f32),
        hdst=pltpu.VMEM((N_D,), f32),
        red=pltpu.VMEM((NSUB, 2, DEG_PER_TILE), f32),
        res=pltpu.VMEM((2, DEG_PER_TILE), f32),
        stage=pltpu.VMEM_SHARED((NSUB, 2, N_D), f32),
    ),
)
def _sc_degrees(ei, do0, do1, di0, di1, spad, dpad,
                esrc, edst, tail, hsrc, hdst, red, res, stage):
  cid = lax.axis_index("c")
  sid = lax.axis_index("s")
  wid = sid * NCORE + cid

  zeros16 = jnp.zeros((LANES,), f32)

  def zero_hist(i, _):
    hsrc[pl.ds(i * LANES, LANES)] = zeros16
    hdst[pl.ds(i * LANES, LANES)] = zeros16
    return 0
  lax.fori_loop(0, N_D // LANES, zero_hist, 0)

  base = wid * E_REAL_PER_TILE
  pltpu.sync_copy(ei.at[0, pl.ds(base, E_REAL_PER_TILE)], esrc)
  pltpu.sync_copy(ei.at[1, pl.ds(base, E_REAL_PER_TILE)], edst)

  dummy16 = jnp.full((LANES,), DUMMY, jnp.int32)
  for i in range(E_TAIL // LANES):
    tail[pl.ds(i * LANES, LANES)] = dummy16
  obase = wid * EDGES_PER_TILE32
  pltpu.sync_copy(esrc, spad.at[pl.ds(obase, E_REAL_PER_TILE)])
  pltpu.sync_copy(edst, dpad.at[pl.ds(obase, E_REAL_PER_TILE)])
  pltpu.sync_copy(tail, spad.at[pl.ds(obase + E_REAL_PER_TILE, E_TAIL)])
  pltpu.sync_copy(tail, dpad.at[pl.ds(obase + E_REAL_PER_TILE, E_TAIL)])

  ones16 = jnp.ones((LANES,), f32)

  def scatter_edges(i, _):
    off = i * LANES
    plsc.addupdate_scatter(hsrc, [esrc[pl.ds(off, LANES)]], ones16)
    plsc.addupdate_scatter(hdst, [edst[pl.ds(off, LANES)]], ones16)
    return 0
  lax.fori_loop(0, E_REAL_PER_TILE // LANES, scatter_edges, 0)

  pltpu.sync_copy(hsrc, stage.at[sid, 0])
  pltpu.sync_copy(hdst, stage.at[sid, 1])
  plsc.subcore_barrier()

  nbase = sid * DEG_PER_TILE
  pltpu.sync_copy(stage.at[:, :, pl.ds(nbase, DEG_PER_TILE)], red)

  def reduce_chunk(n, _):
    off = n * LANES
    for w in range(2):
      acc = red[0, w, pl.ds(off, LANES)]
      for t in range(1, NSUB):
        acc = acc + red[t, w, pl.ds(off, LANES)]
      res[w, pl.ds(off, LANES)] = acc
    return 0
  lax.fori_loop(0, DEG_PER_TILE // LANES, reduce_chunk, 0)

  rows = pl.ds(nbase, DEG_PER_TILE)

  @pl.when(cid == 0)
  def _():
    pltpu.sync_copy(res.at[0], do0.at[rows])
    pltpu.sync_copy(res.at[1], di0.at[rows])

  @pl.when(cid == 1)
  def _():
    pltpu.sync_copy(res.at[0], do1.at[rows])
    pltpu.sync_copy(res.at[1], di1.at[rows])


def _make_sc_agg(D, split32, NBUF, LA):
  ntile = 32 if split32 else 16
  nchunk = E_PAD // (ntile * CHUNK)
  NG = nchunk // NBUF
  ZROWS = ROWS_PER_TILE // 8

  @functools.partial(
      pl.kernel,
      mesh=_mesh,
      compiler_params=pltpu.CompilerParams(use_tc_tiling_on_sc=False),
      out_type=jax.ShapeDtypeStruct((N_T, 128), f32),
      scratch_types=dict(
          es=pltpu.VMEM((3, NBUF, CHUNK), jnp.int32),
          ed=pltpu.VMEM((3, NBUF, CHUNK), jnp.int32),
          gbuf=pltpu.VMEM((NBUF, CHUNK, D), f32),
          zbuf=pltpu.VMEM((ZROWS, D), f32),
          gsem=pltpu.SemaphoreType.DMA((NBUF,)),
          ssem=pltpu.SemaphoreType.DMA((NBUF,)),
          lsem=pltpu.SemaphoreType.DMA((3,)),
          acc=pltpu.VMEM_SHARED((N_T, D), f32),
          tab=pltpu.VMEM_SHARED((N_T, D), f32),
      ),
  )
  def agg(xs, src3, dst3, pout, es, ed, gbuf, zbuf, gsem, ssem,
          lsem, acc, tab):
    cid = lax.axis_index("c")
    sid = lax.axis_index("s")
    wid = sid * NCORE + cid if split32 else sid

    myrows = pl.ds(sid * ROWS_PER_TILE, ROWS_PER_TILE)
    scol = pl.ds(0, D) if split32 else pl.ds(cid * D, D)
    pltpu.sync_copy(xs.at[myrows, scol], tab.at[myrows])

    zeros16 = jnp.zeros((LANES,), f32)

    def zero_rows(j, _):
      for k in range(D // LANES):
        zbuf[j, pl.ds(k * LANES, LANES)] = zeros16
      return 0
    lax.fori_loop(0, ZROWS, zero_rows, 0)
    for z in range(ROWS_PER_TILE // ZROWS):
      pltpu.sync_copy(
          zbuf, acc.at[pl.ds(sid * ROWS_PER_TILE + z * ZROWS, ZROWS)])

    def start_load(g, slot):
      base = g * NBUF
      pltpu.async_copy(src3.at[wid, pl.ds(base, NBUF)], es.at[slot],
                       lsem.at[slot])
      pltpu.async_copy(dst3.at[wid, pl.ds(base, NBUF)], ed.at[slot],
                       lsem.at[slot])

    def wait_load(slot):
      pltpu.make_async_copy(src3.at[wid, pl.ds(0, NBUF)], es.at[slot],
                            lsem.at[slot]).wait()
      pltpu.make_async_copy(dst3.at[wid, pl.ds(0, NBUF)], ed.at[slot],
                            lsem.at[slot]).wait()

    def start_gather(idxref, slot):
      pltpu.async_copy(tab.at[idxref], gbuf.at[slot], gsem.at[slot])

    def wait_gather(slot):
      pltpu.make_async_copy(
          tab.at[es.at[0, 0]], gbuf.at[slot], gsem.at[slot]).wait()

    def wait_scatter(slot):
      pltpu.make_async_copy(
          gbuf.at[slot], acc.at[ed.at[0, 0]], ssem.at[slot]).wait()

    start_load(0, 0)
    wait_load(0)
    start_load(1, 1)
    plsc.subcore_barrier()
    for b in range(LA):
      start_gather(es.at[0, b], b)

    def group_body(g, _):
      cur = lax.rem(g, 3)
      nxt = lax.rem(g + 1, 3)

      @pl.when(g + 1 < NG)
      def _():
        wait_load(nxt)

      for b in range(NBUF):
        j = g * NBUF + b
        jn = j + LA
        slot_n = (b + LA) % NBUF

        @pl.when(jn < nchunk)
        def _():
          @pl.when(j >= LA)
          def _():
            wait_scatter(slot_n)
          if b < NBUF - LA:
            start_gather(es.at[cur, b + LA], slot_n)
          else:
            start_gather(es.at[nxt, b + LA - NBUF], slot_n)

        wait_gather(b)
        pltpu.async_copy(gbuf.at[b], acc.at[ed.at[cur, b]], ssem.at[b],
                         add=True)

      @pl.when(g + 2 < NG)
      def _():
        start_load(g + 2, lax.rem(g + 2, 3))
      return 0
    lax.fori_loop(0, NG, group_body, 0)

    for b in range(NBUF):
      wait_scatter(b)

    plsc.subcore_barrier()

    rows = pl.ds(sid * ROWS_PER_TILE, ROWS_PER_TILE)
    ocol = pl.ds(cid * D, D)
    pltpu.sync_copy(acc.at[rows], pout.at[rows, ocol])

  return agg


_sc_agg64 = _make_sc_agg(64, split32=False, NBUF=4, LA=2)
_sc_agg48 = _make_sc_agg(48, split32=True, NBUF=8, LA=4)


def _norm(d):
  return lax.rsqrt(jnp.where(d > 0.0, d, 1.0))


def _prescale_body(nf, do_c, di_c, xs, nsb, ndb):
  ns = _norm(do_c[...])
  nd = _norm(di_c[...])
  x = nf[...]
  xs[...] = x * ns
  nsb[...] = jnp.broadcast_to(ns, x.shape)
  ndb[...] = jnp.broadcast_to(nd, x.shape)


def _layer_body(a1, ndb, nsb, w1, b1, w2, out):
  h = jnp.dot(a1[...] * ndb[...], w1[...], preferred_element_type=f32)
  h = jax.nn.relu(h + b1[...])
  out[...] = jnp.dot(h, w2[...], preferred_element_type=f32) * nsb[...]


def _final_body(pp, ndb, b2, out):
  p = pp[...]
  out[...] = (p[:, :C] + p[:, 48:48 + C]) * ndb[:, :C] + b2[...]


def _row_spec(width):
  return pl.BlockSpec((R_BLK, width), lambda r: (r, 0))


def _full_spec(shape):
  return pl.BlockSpec(shape, lambda r: tuple(0 for _ in shape))


_tc_prescale = pl.pallas_call(
    _prescale_body,
    grid=(GRID_R,),
    in_specs=[_row_spec(128), _row_spec(1), _row_spec(1)],
    out_specs=[_row_spec(128)] * 3,
    out_shape=[jax.ShapeDtypeStruct((N_T, 128), f32)] * 3,
)

_tc_layer = pl.pallas_call(
    _layer_body,
    grid=(GRID_R,),
    in_specs=[_row_spec(128), _row_spec(128), _row_spec(128),
              _full_spec((128, 128)), _full_spec((1, 128)),
              _full_spec((128, 128))],
    out_specs=[_row_spec(128)],
    out_shape=[jax.ShapeDtypeStruct((N_T, 128), f32)],
)

F_BLK = 2000
_tc_final = pl.pallas_call(
    _final_body,
    grid=(N // F_BLK,),
    in_specs=[pl.BlockSpec((F_BLK, 128), lambda r: (r, 0)),
              pl.BlockSpec((F_BLK, 128), lambda r: (r, 0)),
              _full_spec((1, C))],
    out_specs=[pl.BlockSpec((F_BLK, C), lambda r: (r, 0))],
    out_shape=[jax.ShapeDtypeStruct((N, C), f32)],
)


def kernel(n_feats, edge_index, W1, b1, W2, b2):
  do0, do1, di0, di1, srcp, dstp = _sc_degrees(edge_index)
  src16 = srcp.reshape(16, E_PAD // (16 * CHUNK), CHUNK)
  dst16 = dstp.reshape(16, E_PAD // (16 * CHUNK), CHUNK)
  src32 = srcp.reshape(32, E_PAD // (32 * CHUNK), CHUNK)
  dst32 = dstp.reshape(32, E_PAD // (32 * CHUNK), CHUNK)
  do_c = (do0 + do1).reshape(N_D, 1)
  di_c = (di0 + di1).reshape(N_D, 1)

  xs, nsb, ndb = _tc_prescale(n_feats, do_c, di_c)
  a1 = _sc_agg64(xs, src16, dst16)

  w2p = jnp.pad(W2, ((0, 0), (0, 128 - C)))
  (t2,) = _tc_layer(a1, ndb, nsb, W1, b1.reshape(1, H), w2p)

  pp = _sc_agg48(t2, src32, dst32)
  (outp,) = _tc_final(pp, ndb, b2.reshape(1, C))
  return outp


# speedup vs baseline: 1.0555x; 1.0145x over previous
import functools

import jax
import jax.numpy as jnp
from jax import lax
from jax.experimental import pallas as pl
from jax.experimental.pallas import tpu as pltpu
from jax.experimental.pallas import tpu_sc as plsc

N = 10000
E = 320000
F = 128
H = 128
C = 40

NCORE = 2
NSUB = 16
LANES = 16

DUMMY = N
N_T = 10112
ROWS_PER_TILE = N_T // NSUB
N_D = 10240
DEG_PER_TILE = N_D // NSUB
E_PAD = 327680
EDGES_PER_TILE32 = E_PAD // 32
CHUNK = 128

R_BLK = 2528
GRID_R = N_T // R_BLK

_mesh = plsc.VectorSubcoreMesh(
    core_axis_name="c", subcore_axis_name="s", num_cores=NCORE,
    num_subcores=NSUB)

f32 = jnp.float32


E_REAL_PER_TILE = E // 32
E_TAIL = EDGES_PER_TILE32 - E_REAL_PER_TILE


@functools.partial(
    pl.kernel,
    mesh=_mesh,
    compiler_params=pltpu.CompilerParams(
        needs_layout_passes=False, use_tc_tiling_on_sc=False),
    out_type=(
        jax.ShapeDtypeStruct((N_D,), f32),
        jax.ShapeDtypeStruct((N_D,), f32),
        jax.ShapeDtypeStruct((N_D,), f32),
        jax.ShapeDtypeStruct((N_D,), f32),
        jax.ShapeDtypeStruct((E_PAD,), jnp.int32),
        jax.ShapeDtypeStruct((E_PAD,), jnp.int32),
    ),
    scratch_types=dict(
        esrc=pltpu.VMEM((E_REAL_PER_TILE,), jnp.int32),
        edst=pltpu.VMEM((E_REAL_PER_TILE,), jnp.int32),
        tail=pltpu.VMEM((E_TAIL,), jnp.int32),
        hsrc=pltpu.VMEM((N_D,), f32),
        hdst=pltpu.VMEM((N_D,), f32),
        red=pltpu.VMEM((NSUB, 2, DEG_PER_TILE), f32),
        res=pltpu.VMEM((2, DEG_PER_TILE), f32),
        stage=pltpu.VMEM_SHARED((NSUB, 2, N_D), f32),
    ),
)
def _sc_degrees(ei, do0, do1, di0, di1, spad, dpad,
                esrc, edst, tail, hsrc, hdst, red, res, stage):
  cid = lax.axis_index("c")
  sid = lax.axis_index("s")
  wid = sid * NCORE + cid

  zeros16 = jnp.zeros((LANES,), f32)

  def zero_hist(i, _):
    hsrc[pl.ds(i * LANES, LANES)] = zeros16
    hdst[pl.ds(i * LANES, LANES)] = zeros16
    return 0
  lax.fori_loop(0, N_D // LANES, zero_hist, 0)

  base = wid * E_REAL_PER_TILE
  pltpu.sync_copy(ei.at[0, pl.ds(base, E_REAL_PER_TILE)], esrc)
  pltpu.sync_copy(ei.at[1, pl.ds(base, E_REAL_PER_TILE)], edst)

  dummy16 = jnp.full((LANES,), DUMMY, jnp.int32)
  for i in range(E_TAIL // LANES):
    tail[pl.ds(i * LANES, LANES)] = dummy16
  obase = wid * EDGES_PER_TILE32
  pltpu.sync_copy(esrc, spad.at[pl.ds(obase, E_REAL_PER_TILE)])
  pltpu.sync_copy(edst, dpad.at[pl.ds(obase, E_REAL_PER_TILE)])
  pltpu.sync_copy(tail, spad.at[pl.ds(obase + E_REAL_PER_TILE, E_TAIL)])
  pltpu.sync_copy(tail, dpad.at[pl.ds(obase + E_REAL_PER_TILE, E_TAIL)])

  ones16 = jnp.ones((LANES,), f32)

  def scatter_edges(i, _):
    off = i * LANES
    plsc.addupdate_scatter(hsrc, [esrc[pl.ds(off, LANES)]], ones16)
    plsc.addupdate_scatter(hdst, [edst[pl.ds(off, LANES)]], ones16)
    return 0
  lax.fori_loop(0, E_REAL_PER_TILE // LANES, scatter_edges, 0)

  pltpu.sync_copy(hsrc, stage.at[sid, 0])
  pltpu.sync_copy(hdst, stage.at[sid, 1])
  plsc.subcore_barrier()

  nbase = sid * DEG_PER_TILE
  pltpu.sync_copy(stage.at[:, :, pl.ds(nbase, DEG_PER_TILE)], red)

  def reduce_chunk(n, _):
    off = n * LANES
    for w in range(2):
      acc = red[0, w, pl.ds(off, LANES)]
      for t in range(1, NSUB):
        acc = acc + red[t, w, pl.ds(off, LANES)]
      res[w, pl.ds(off, LANES)] = acc
    return 0
  lax.fori_loop(0, DEG_PER_TILE // LANES, reduce_chunk, 0)

  rows = pl.ds(nbase, DEG_PER_TILE)

  @pl.when(cid == 0)
  def _():
    pltpu.sync_copy(res.at[0], do0.at[rows])
    pltpu.sync_copy(res.at[1], di0.at[rows])

  @pl.when(cid == 1)
  def _():
    pltpu.sync_copy(res.at[0], do1.at[rows])
    pltpu.sync_copy(res.at[1], di1.at[rows])


def _make_sc_agg(D, split32, NBUF, LA):
  ntile = 32 if split32 else 16
  nchunk = E_PAD // (ntile * CHUNK)
  NG = nchunk // NBUF
  ZROWS = ROWS_PER_TILE // 8

  @functools.partial(
      pl.kernel,
      mesh=_mesh,
      compiler_params=pltpu.CompilerParams(use_tc_tiling_on_sc=False),
      out_type=jax.ShapeDtypeStruct((N_T, 128), f32),
      scratch_types=dict(
          es=pltpu.VMEM((3, NBUF, CHUNK), jnp.int32),
          ed=pltpu.VMEM((3, NBUF, CHUNK), jnp.int32),
          gbuf=pltpu.VMEM((NBUF, CHUNK, D), f32),
          zbuf=pltpu.VMEM((ZROWS, D), f32),
          gsem=pltpu.SemaphoreType.DMA((NBUF,)),
          ssem=pltpu.SemaphoreType.DMA((NBUF,)),
          lsem=pltpu.SemaphoreType.DMA((3,)),
          acc=pltpu.VMEM_SHARED((N_T, D), f32),
          tab=pltpu.VMEM_SHARED((N_T, D), f32),
      ),
  )
  def agg(xs, src3, dst3, pout, es, ed, gbuf, zbuf, gsem, ssem,
          lsem, acc, tab):
    cid = lax.axis_index("c")
    sid = lax.axis_index("s")
    wid = sid * NCORE + cid if split32 else sid

    myrows = pl.ds(sid * ROWS_PER_TILE, ROWS_PER_TILE)
    scol = pl.ds(0, D) if split32 else pl.ds(cid * D, D)
    pltpu.sync_copy(xs.at[myrows, scol], tab.at[myrows])

    zeros16 = jnp.zeros((LANES,), f32)

    def zero_rows(j, _):
      for k in range(D // LANES):
        zbuf[j, pl.ds(k * LANES, LANES)] = zeros16
      return 0
    lax.fori_loop(0, ZROWS, zero_rows, 0)
    for z in range(ROWS_PER_TILE // ZROWS):
      pltpu.sync_copy(
          zbuf, acc.at[pl.ds(sid * ROWS_PER_TILE + z * ZROWS, ZROWS)])

    def start_load(g, slot):
      base = g * NBUF
      pltpu.async_copy(src3.at[wid, pl.ds(base, NBUF)], es.at[slot],
                       lsem.at[slot])
      pltpu.async_copy(dst3.at[wid, pl.ds(base, NBUF)], ed.at[slot],
                       lsem.at[slot])

    def wait_load(slot):
      pltpu.make_async_copy(src3.at[wid, pl.ds(0, NBUF)], es.at[slot],
                            lsem.at[slot]).wait()
      pltpu.make_async_copy(dst3.at[wid, pl.ds(0, NBUF)], ed.at[slot],
                            lsem.at[slot]).wait()

    def start_gather(idxref, slot):
      pltpu.async_copy(tab.at[idxref], gbuf.at[slot], gsem.at[slot])

    def wait_gather(slot):
      pltpu.make_async_copy(
          tab.at[es.at[0, 0]], gbuf.at[slot], gsem.at[slot]).wait()

    def wait_scatter(slot):
      pltpu.make_async_copy(
          gbuf.at[slot], acc.at[ed.at[0, 0]], ssem.at[slot]).wait()

    start_load(0, 0)
    wait_load(0)
    start_load(1, 1)
    plsc.subcore_barrier()
    for b in range(LA):
      start_gather(es.at[0, b], b)

    def group_body(g, _):
      cur = lax.rem(g, 3)
      nxt = lax.rem(g + 1, 3)

      @pl.when(g + 1 < NG)
      def _():
        wait_load(nxt)

      for b in range(NBUF):
        j = g * NBUF + b
        jn = j + LA
        slot_n = (b + LA) % NBUF

        @pl.when(jn < nchunk)
        def _():
          @pl.when(j >= LA)
          def _():
            wait_scatter(slot_n)
          if b < NBUF - LA:
            start_gather(es.at[cur, b + LA], slot_n)
          else:
            start_gather(es.at[nxt, b + LA - NBUF], slot_n)

        wait_gather(b)
        pltpu.async_copy(gbuf.at[b], acc.at[ed.at[cur, b]], ssem.at[b],
                         add=True)

      @pl.when(g + 2 < NG)
      def _():
        start_load(g + 2, lax.rem(g + 2, 3))
      return 0
    lax.fori_loop(0, NG, group_body, 0)

    for b in range(NBUF):
      wait_scatter(b)

    plsc.subcore_barrier()

    rows = pl.ds(sid * ROWS_PER_TILE, ROWS_PER_TILE)
    ocol = pl.ds(cid * D, D)
    pltpu.sync_copy(acc.at[rows], pout.at[rows, ocol])

  return agg


_sc_agg64 = _make_sc_agg(64, split32=False, NBUF=4, LA=2)
_sc_agg48 = _make_sc_agg(48, split32=True, NBUF=8, LA=4)


def _norm(d):
  return lax.rsqrt(jnp.where(d > 0.0, d, 1.0))


def _prescale_body(nf, do_c, di_c, xs, nsb, ndb):
  ns = _norm(do_c[...])
  nd = _norm(di_c[...])
  x = nf[...]
  xs[...] = x * ns
  nsb[...] = jnp.broadcast_to(ns.astype(jnp.bfloat16), x.shape)
  ndb[...] = jnp.broadcast_to(nd.astype(jnp.bfloat16), x.shape)


def _layer_body(a1, ndb, nsb, w1, b1, w2, out):
  nd = ndb[...].astype(f32)
  h = jnp.dot(a1[...] * nd, w1[...], preferred_element_type=f32)
  h = jax.nn.relu(h + b1[...])
  ns = nsb[...].astype(f32)
  out[...] = jnp.dot(h, w2[...], preferred_element_type=f32) * ns


def _final_body(pp, ndb, b2, out):
  p = pp[...]
  nd = ndb[...].astype(f32)
  out[...] = (p[:, :C] + p[:, 48:48 + C]) * nd[:, :C] + b2[...]


def _row_spec(width):
  return pl.BlockSpec((R_BLK, width), lambda r: (r, 0))


def _full_spec(shape):
  return pl.BlockSpec(shape, lambda r: tuple(0 for _ in shape))


_tc_prescale = pl.pallas_call(
    _prescale_body,
    grid=(GRID_R,),
    in_specs=[_row_spec(128), _row_spec(1), _row_spec(1)],
    out_specs=[_row_spec(128)] * 3,
    out_shape=[jax.ShapeDtypeStruct((N_T, 128), f32),
               jax.ShapeDtypeStruct((N_T, 128), jnp.bfloat16),
               jax.ShapeDtypeStruct((N_T, 128), jnp.bfloat16)],
)

_tc_layer = pl.pallas_call(
    _layer_body,
    grid=(GRID_R,),
    in_specs=[_row_spec(128), _row_spec(128), _row_spec(128),
              _full_spec((128, 128)), _full_spec((1, 128)),
              _full_spec((128, 128))],
    out_specs=[_row_spec(128)],
    out_shape=[jax.ShapeDtypeStruct((N_T, 128), f32)],
)

F_BLK = 2000
_tc_final = pl.pallas_call(
    _final_body,
    grid=(N // F_BLK,),
    in_specs=[pl.BlockSpec((F_BLK, 128), lambda r: (r, 0)),
              pl.BlockSpec((F_BLK, 128), lambda r: (r, 0)),
              _full_spec((1, C))],
    out_specs=[pl.BlockSpec((F_BLK, C), lambda r: (r, 0))],
    out_shape=[jax.ShapeDtypeStruct((N, C), f32)],
)


def kernel(n_feats, edge_index, W1, b1, W2, b2):
  do0, do1, di0, di1, srcp, dstp = _sc_degrees(edge_index)
  src16 = srcp.reshape(16, E_PAD // (16 * CHUNK), CHUNK)
  dst16 = dstp.reshape(16, E_PAD // (16 * CHUNK), CHUNK)
  src32 = srcp.reshape(32, E_PAD // (32 * CHUNK), CHUNK)
  dst32 = dstp.reshape(32, E_PAD // (32 * CHUNK), CHUNK)
  do_c = (do0 + do1).reshape(N_D, 1)
  di_c = (di0 + di1).reshape(N_D, 1)

  xs, nsb, ndb = _tc_prescale(n_feats, do_c, di_c)
  a1 = _sc_agg64(xs, src16, dst16)

  w2p = jnp.pad(W2, ((0, 0), (0, 128 - C)))
  (t2,) = _tc_layer(a1, ndb, nsb, W1, b1.reshape(1, H), w2p)

  pp = _sc_agg48(t2, src32, dst32)
  (outp,) = _tc_final(pp, ndb, b2.reshape(1, C))
  return outp


# speedup vs baseline: 1.0818x; 1.0249x over previous
import functools

import jax
import jax.numpy as jnp
from jax import lax
from jax.experimental import pallas as pl
from jax.experimental.pallas import tpu as pltpu
from jax.experimental.pallas import tpu_sc as plsc

N = 10000
E = 320000
F = 128
H = 128
C = 40

NCORE = 2
NSUB = 16
LANES = 16

DUMMY = N
N_T = 10112
ROWS_PER_TILE = N_T // NSUB
N_D = 10240
DEG_PER_TILE = N_D // NSUB
E_PAD = 327680
EDGES_PER_TILE32 = E_PAD // 32
CHUNK = 128

R_BLK = 2528
GRID_R = N_T // R_BLK

_mesh = plsc.VectorSubcoreMesh(
    core_axis_name="c", subcore_axis_name="s", num_cores=NCORE,
    num_subcores=NSUB)

f32 = jnp.float32


E_REAL_PER_TILE = E // 32
E_TAIL = EDGES_PER_TILE32 - E_REAL_PER_TILE


@functools.partial(
    pl.kernel,
    mesh=_mesh,
    compiler_params=pltpu.CompilerParams(
        needs_layout_passes=False, use_tc_tiling_on_sc=False),
    out_type=(
        jax.ShapeDtypeStruct((N_D,), f32),
        jax.ShapeDtypeStruct((N_D,), f32),
        jax.ShapeDtypeStruct((N_D,), f32),
        jax.ShapeDtypeStruct((N_D,), f32),
        jax.ShapeDtypeStruct((E_PAD,), jnp.int32),
        jax.ShapeDtypeStruct((E_PAD,), jnp.int32),
    ),
    scratch_types=dict(
        esrc=pltpu.VMEM((E_REAL_PER_TILE,), jnp.int32),
        edst=pltpu.VMEM((E_REAL_PER_TILE,), jnp.int32),
        tail=pltpu.VMEM((E_TAIL,), jnp.int32),
        hsrc=pltpu.VMEM((N_D,), f32),
        hdst=pltpu.VMEM((N_D,), f32),
        red=pltpu.VMEM((NSUB, 2, DEG_PER_TILE), f32),
        res=pltpu.VMEM((2, DEG_PER_TILE), f32),
        stage=pltpu.VMEM_SHARED((NSUB, 2, N_D), f32),
    ),
)
def _sc_degrees(ei, do0, do1, di0, di1, spad, dpad,
                esrc, edst, tail, hsrc, hdst, red, res, stage):
  cid = lax.axis_index("c")
  sid = lax.axis_index("s")
  wid = sid * NCORE + cid

  zeros16 = jnp.zeros((LANES,), f32)

  def zero_hist(i, _):
    hsrc[pl.ds(i * LANES, LANES)] = zeros16
    hdst[pl.ds(i * LANES, LANES)] = zeros16
    return 0
  lax.fori_loop(0, N_D // LANES, zero_hist, 0)

  base = wid * E_REAL_PER_TILE
  pltpu.sync_copy(ei.at[0, pl.ds(base, E_REAL_PER_TILE)], esrc)
  pltpu.sync_copy(ei.at[1, pl.ds(base, E_REAL_PER_TILE)], edst)

  dummy16 = jnp.full((LANES,), DUMMY, jnp.int32)
  for i in range(E_TAIL // LANES):
    tail[pl.ds(i * LANES, LANES)] = dummy16
  obase = wid * EDGES_PER_TILE32
  pltpu.sync_copy(esrc, spad.at[pl.ds(obase, E_REAL_PER_TILE)])
  pltpu.sync_copy(edst, dpad.at[pl.ds(obase, E_REAL_PER_TILE)])
  pltpu.sync_copy(tail, spad.at[pl.ds(obase + E_REAL_PER_TILE, E_TAIL)])
  pltpu.sync_copy(tail, dpad.at[pl.ds(obase + E_REAL_PER_TILE, E_TAIL)])

  ones16 = jnp.ones((LANES,), f32)

  def scatter_edges(i, _):
    off = i * LANES
    plsc.addupdate_scatter(hsrc, [esrc[pl.ds(off, LANES)]], ones16)
    plsc.addupdate_scatter(hdst, [edst[pl.ds(off, LANES)]], ones16)
    return 0
  lax.fori_loop(0, E_REAL_PER_TILE // LANES, scatter_edges, 0)

  pltpu.sync_copy(hsrc, stage.at[sid, 0])
  pltpu.sync_copy(hdst, stage.at[sid, 1])
  plsc.subcore_barrier()

  nbase = sid * DEG_PER_TILE
  pltpu.sync_copy(stage.at[:, :, pl.ds(nbase, DEG_PER_TILE)], red)

  def reduce_chunk(n, _):
    off = n * LANES
    for w in range(2):
      acc = red[0, w, pl.ds(off, LANES)]
      for t in range(1, NSUB):
        acc = acc + red[t, w, pl.ds(off, LANES)]
      res[w, pl.ds(off, LANES)] = acc
    return 0
  lax.fori_loop(0, DEG_PER_TILE // LANES, reduce_chunk, 0)

  rows = pl.ds(nbase, DEG_PER_TILE)

  @pl.when(cid == 0)
  def _():
    pltpu.sync_copy(res.at[0], do0.at[rows])
    pltpu.sync_copy(res.at[1], di0.at[rows])

  @pl.when(cid == 1)
  def _():
    pltpu.sync_copy(res.at[0], do1.at[rows])
    pltpu.sync_copy(res.at[1], di1.at[rows])


def _make_sc_agg(D, split32, NBUF, LA):
  ntile = 32 if split32 else 16
  nchunk = E_PAD // (ntile * CHUNK)
  NG = nchunk // NBUF
  ZROWS = ROWS_PER_TILE // 8

  @functools.partial(
      pl.kernel,
      mesh=_mesh,
      compiler_params=pltpu.CompilerParams(use_tc_tiling_on_sc=False),
      out_type=jax.ShapeDtypeStruct((N_T, 128), f32),
      scratch_types=dict(
          es=pltpu.VMEM((3, NBUF, CHUNK), jnp.int32),
          ed=pltpu.VMEM((3, NBUF, CHUNK), jnp.int32),
          gbuf=pltpu.VMEM((NBUF, CHUNK, D), f32),
          zbuf=pltpu.VMEM((ZROWS, D), f32),
          gsem=pltpu.SemaphoreType.DMA((NBUF,)),
          ssem=pltpu.SemaphoreType.DMA((NBUF,)),
          lsem=pltpu.SemaphoreType.DMA((3,)),
          acc=pltpu.VMEM_SHARED((N_T, D), f32),
          tab=pltpu.VMEM_SHARED((N_T, D), f32),
      ),
  )
  def agg(xs, src3, dst3, pout, es, ed, gbuf, zbuf, gsem, ssem,
          lsem, acc, tab):
    cid = lax.axis_index("c")
    sid = lax.axis_index("s")
    wid = sid * NCORE + cid if split32 else sid

    myrows = pl.ds(sid * ROWS_PER_TILE, ROWS_PER_TILE)
    scol = pl.ds(0, D) if split32 else pl.ds(cid * D, D)
    pltpu.sync_copy(xs.at[myrows, scol], tab.at[myrows])

    zeros16 = jnp.zeros((LANES,), f32)

    def zero_rows(j, _):
      for k in range(D // LANES):
        zbuf[j, pl.ds(k * LANES, LANES)] = zeros16
      return 0
    lax.fori_loop(0, ZROWS, zero_rows, 0)
    for z in range(ROWS_PER_TILE // ZROWS):
      pltpu.sync_copy(
          zbuf, acc.at[pl.ds(sid * ROWS_PER_TILE + z * ZROWS, ZROWS)])

    def start_load(g, slot):
      base = g * NBUF
      pltpu.async_copy(src3.at[wid, pl.ds(base, NBUF)], es.at[slot],
                       lsem.at[slot])
      pltpu.async_copy(dst3.at[wid, pl.ds(base, NBUF)], ed.at[slot],
                       lsem.at[slot])

    def wait_load(slot):
      pltpu.make_async_copy(src3.at[wid, pl.ds(0, NBUF)], es.at[slot],
                            lsem.at[slot]).wait()
      pltpu.make_async_copy(dst3.at[wid, pl.ds(0, NBUF)], ed.at[slot],
                            lsem.at[slot]).wait()

    def start_gather(idxref, slot):
      pltpu.async_copy(tab.at[idxref], gbuf.at[slot], gsem.at[slot])

    def wait_gather(slot):
      pltpu.make_async_copy(
          tab.at[es.at[0, 0]], gbuf.at[slot], gsem.at[slot]).wait()

    def wait_scatter(slot):
      pltpu.make_async_copy(
          gbuf.at[slot], acc.at[ed.at[0, 0]], ssem.at[slot]).wait()

    start_load(0, 0)
    wait_load(0)
    start_load(1, 1)
    plsc.subcore_barrier()
    for b in range(LA):
      start_gather(es.at[0, b], b)

    def group_body(g, _):
      cur = lax.rem(g, 3)
      nxt = lax.rem(g + 1, 3)

      @pl.when(g + 1 < NG)
      def _():
        wait_load(nxt)

      for b in range(NBUF):
        j = g * NBUF + b
        jn = j + LA
        slot_n = (b + LA) % NBUF

        @pl.when(jn < nchunk)
        def _():
          @pl.when(j >= LA)
          def _():
            wait_scatter(slot_n)
          if b < NBUF - LA:
            start_gather(es.at[cur, b + LA], slot_n)
          else:
            start_gather(es.at[nxt, b + LA - NBUF], slot_n)

        wait_gather(b)
        pltpu.async_copy(gbuf.at[b], acc.at[ed.at[cur, b]], ssem.at[b],
                         add=True)

      @pl.when(g + 2 < NG)
      def _():
        start_load(g + 2, lax.rem(g + 2, 3))
      return 0
    lax.fori_loop(0, NG, group_body, 0)

    for b in range(NBUF):
      wait_scatter(b)

    plsc.subcore_barrier()

    rows = pl.ds(sid * ROWS_PER_TILE, ROWS_PER_TILE)
    ocol = pl.ds(cid * D, D)
    pltpu.sync_copy(acc.at[rows], pout.at[rows, ocol])

  return agg


_sc_agg64 = _make_sc_agg(64, split32=False, NBUF=4, LA=2)
_sc_agg48 = _make_sc_agg(40, split32=True, NBUF=8, LA=4)


def _norm(d):
  return lax.rsqrt(jnp.where(d > 0.0, d, 1.0))


def _prescale_body(nf, do_c, di_c, xs, nsb, ndb):
  ns = _norm(do_c[...])
  nd = _norm(di_c[...])
  x = nf[...]
  xs[...] = x * ns
  nsb[...] = jnp.broadcast_to(ns.astype(jnp.bfloat16), x.shape)
  ndb[...] = jnp.broadcast_to(nd.astype(jnp.bfloat16), x.shape)


def _layer_body(a1, ndb, nsb, w1, b1, w2, out):
  nd = ndb[...].astype(f32)
  h = jnp.dot(a1[...] * nd, w1[...], preferred_element_type=f32)
  h = jax.nn.relu(h + b1[...])
  ns = nsb[...].astype(f32)
  out[...] = jnp.dot(h, w2[...], preferred_element_type=f32) * ns


def _final_body(pp, ndb, b2, out):
  p = pp[...]
  nd = ndb[...].astype(f32)
  out[...] = (p[:, :C] + p[:, C:2 * C]) * nd[:, :C] + b2[...]


def _row_spec(width):
  return pl.BlockSpec((R_BLK, width), lambda r: (r, 0))


def _full_spec(shape):
  return pl.BlockSpec(shape, lambda r: tuple(0 for _ in shape))


_tc_prescale = pl.pallas_call(
    _prescale_body,
    grid=(GRID_R,),
    in_specs=[_row_spec(128), _row_spec(1), _row_spec(1)],
    out_specs=[_row_spec(128)] * 3,
    out_shape=[jax.ShapeDtypeStruct((N_T, 128), f32),
               jax.ShapeDtypeStruct((N_T, 128), jnp.bfloat16),
               jax.ShapeDtypeStruct((N_T, 128), jnp.bfloat16)],
)

_tc_layer = pl.pallas_call(
    _layer_body,
    grid=(GRID_R,),
    in_specs=[_row_spec(128), _row_spec(128), _row_spec(128),
              _full_spec((128, 128)), _full_spec((1, 128)),
              _full_spec((128, 128))],
    out_specs=[_row_spec(128)],
    out_shape=[jax.ShapeDtypeStruct((N_T, 128), f32)],
)

F_BLK = 2000
_tc_final = pl.pallas_call(
    _final_body,
    grid=(N // F_BLK,),
    in_specs=[pl.BlockSpec((F_BLK, 128), lambda r: (r, 0)),
              pl.BlockSpec((F_BLK, 128), lambda r: (r, 0)),
              _full_spec((1, C))],
    out_specs=[pl.BlockSpec((F_BLK, C), lambda r: (r, 0))],
    out_shape=[jax.ShapeDtypeStruct((N, C), f32)],
)


def kernel(n_feats, edge_index, W1, b1, W2, b2):
  do0, do1, di0, di1, srcp, dstp = _sc_degrees(edge_index)
  src16 = srcp.reshape(16, E_PAD // (16 * CHUNK), CHUNK)
  dst16 = dstp.reshape(16, E_PAD // (16 * CHUNK), CHUNK)
  src32 = srcp.reshape(32, E_PAD // (32 * CHUNK), CHUNK)
  dst32 = dstp.reshape(32, E_PAD // (32 * CHUNK), CHUNK)
  do_c = (do0 + do1).reshape(N_D, 1)
  di_c = (di0 + di1).reshape(N_D, 1)

  xs, nsb, ndb = _tc_prescale(n_feats, do_c, di_c)
  a1 = _sc_agg64(xs, src16, dst16)

  w2p = jnp.pad(W2, ((0, 0), (0, 128 - C)))
  (t2,) = _tc_layer(a1, ndb, nsb, W1, b1.reshape(1, H), w2p)

  pp = _sc_agg48(t2, src32, dst32)
  (outp,) = _tc_final(pp, ndb, b2.reshape(1, C))
  return outp


# speedup vs baseline: 1.0820x; 1.0002x over previous
import functools

import jax
import jax.numpy as jnp
from jax import lax
from jax.experimental import pallas as pl
from jax.experimental.pallas import tpu as pltpu
from jax.experimental.pallas import tpu_sc as plsc

N = 10000
E = 320000
F = 128
H = 128
C = 40

NCORE = 2
NSUB = 16
LANES = 16

DUMMY = N
N_T = 10112
ROWS_PER_TILE = N_T // NSUB
N_D = 10240
DEG_PER_TILE = N_D // NSUB
E_PAD = 327680
EDGES_PER_TILE32 = E_PAD // 32
CHUNK = 128

R_BLK = 2528
GRID_R = N_T // R_BLK

_mesh = plsc.VectorSubcoreMesh(
    core_axis_name="c", subcore_axis_name="s", num_cores=NCORE,
    num_subcores=NSUB)

f32 = jnp.float32


E_REAL_PER_TILE = E // 32
E_TAIL = EDGES_PER_TILE32 - E_REAL_PER_TILE


@functools.partial(
    pl.kernel,
    mesh=_mesh,
    compiler_params=pltpu.CompilerParams(
        needs_layout_passes=False, use_tc_tiling_on_sc=False),
    out_type=(
        jax.ShapeDtypeStruct((N_D,), f32),
        jax.ShapeDtypeStruct((N_D,), f32),
        jax.ShapeDtypeStruct((N_D,), f32),
        jax.ShapeDtypeStruct((N_D,), f32),
        jax.ShapeDtypeStruct((E_PAD,), jnp.int32),
        jax.ShapeDtypeStruct((E_PAD,), jnp.int32),
    ),
    scratch_types=dict(
        esrc=pltpu.VMEM((E_REAL_PER_TILE,), jnp.int32),
        edst=pltpu.VMEM((E_REAL_PER_TILE,), jnp.int32),
        tail=pltpu.VMEM((E_TAIL,), jnp.int32),
        hsrc=pltpu.VMEM((N_D,), f32),
        hdst=pltpu.VMEM((N_D,), f32),
        red=pltpu.VMEM((NSUB, 2, DEG_PER_TILE), f32),
        res=pltpu.VMEM((2, DEG_PER_TILE), f32),
        stage=pltpu.VMEM_SHARED((NSUB, 2, N_D), f32),
    ),
)
def _sc_degrees(ei, do0, do1, di0, di1, spad, dpad,
                esrc, edst, tail, hsrc, hdst, red, res, stage):
  cid = lax.axis_index("c")
  sid = lax.axis_index("s")
  wid = sid * NCORE + cid

  zeros16 = jnp.zeros((LANES,), f32)

  def zero_hist(i, _):
    hsrc[pl.ds(i * LANES, LANES)] = zeros16
    hdst[pl.ds(i * LANES, LANES)] = zeros16
    return 0
  lax.fori_loop(0, N_D // LANES, zero_hist, 0)

  base = wid * E_REAL_PER_TILE
  pltpu.sync_copy(ei.at[0, pl.ds(base, E_REAL_PER_TILE)], esrc)
  pltpu.sync_copy(ei.at[1, pl.ds(base, E_REAL_PER_TILE)], edst)

  dummy16 = jnp.full((LANES,), DUMMY, jnp.int32)
  for i in range(E_TAIL // LANES):
    tail[pl.ds(i * LANES, LANES)] = dummy16
  obase = wid * EDGES_PER_TILE32
  pltpu.sync_copy(esrc, spad.at[pl.ds(obase, E_REAL_PER_TILE)])
  pltpu.sync_copy(edst, dpad.at[pl.ds(obase, E_REAL_PER_TILE)])
  pltpu.sync_copy(tail, spad.at[pl.ds(obase + E_REAL_PER_TILE, E_TAIL)])
  pltpu.sync_copy(tail, dpad.at[pl.ds(obase + E_REAL_PER_TILE, E_TAIL)])

  ones16 = jnp.ones((LANES,), f32)

  def scatter_edges(i, _):
    off = i * LANES
    plsc.addupdate_scatter(hsrc, [esrc[pl.ds(off, LANES)]], ones16)
    plsc.addupdate_scatter(hdst, [edst[pl.ds(off, LANES)]], ones16)
    return 0
  lax.fori_loop(0, E_REAL_PER_TILE // LANES, scatter_edges, 0)

  pltpu.sync_copy(hsrc, stage.at[sid, 0])
  pltpu.sync_copy(hdst, stage.at[sid, 1])
  plsc.subcore_barrier()

  nbase = sid * DEG_PER_TILE
  pltpu.sync_copy(stage.at[:, :, pl.ds(nbase, DEG_PER_TILE)], red)

  def reduce_chunk(n, _):
    off = n * LANES
    for w in range(2):
      acc = red[0, w, pl.ds(off, LANES)]
      for t in range(1, NSUB):
        acc = acc + red[t, w, pl.ds(off, LANES)]
      res[w, pl.ds(off, LANES)] = acc
    return 0
  lax.fori_loop(0, DEG_PER_TILE // LANES, reduce_chunk, 0)

  rows = pl.ds(nbase, DEG_PER_TILE)

  @pl.when(cid == 0)
  def _():
    pltpu.sync_copy(res.at[0], do0.at[rows])
    pltpu.sync_copy(res.at[1], di0.at[rows])

  @pl.when(cid == 1)
  def _():
    pltpu.sync_copy(res.at[0], do1.at[rows])
    pltpu.sync_copy(res.at[1], di1.at[rows])


def _make_sc_agg(D, split32, NBUF, LA):
  ntile = 32 if split32 else 16
  nchunk = E_PAD // (ntile * CHUNK)
  NG = nchunk // NBUF
  ZROWS = ROWS_PER_TILE // 8

  @functools.partial(
      pl.kernel,
      mesh=_mesh,
      compiler_params=pltpu.CompilerParams(use_tc_tiling_on_sc=False),
      out_type=jax.ShapeDtypeStruct((N_T, 128), f32),
      scratch_types=dict(
          es=pltpu.VMEM((3, NBUF, CHUNK), jnp.int32),
          ed=pltpu.VMEM((3, NBUF, CHUNK), jnp.int32),
          gbuf=pltpu.VMEM((NBUF, CHUNK, D), f32),
          zbuf=pltpu.VMEM((ZROWS, D), f32),
          gsem=pltpu.SemaphoreType.DMA((NBUF,)),
          ssem=pltpu.SemaphoreType.DMA((NBUF,)),
          lsem=pltpu.SemaphoreType.DMA((3,)),
          acc=pltpu.VMEM_SHARED((N_T, D), f32),
          tab=pltpu.VMEM_SHARED((N_T, D), f32),
      ),
  )
  def agg(xs, src3, dst3, pout, es, ed, gbuf, zbuf, gsem, ssem,
          lsem, acc, tab):
    cid = lax.axis_index("c")
    sid = lax.axis_index("s")
    wid = sid * NCORE + cid if split32 else sid

    myrows = pl.ds(sid * ROWS_PER_TILE, ROWS_PER_TILE)
    scol = pl.ds(0, D) if split32 else pl.ds(cid * D, D)
    pltpu.sync_copy(xs.at[myrows, scol], tab.at[myrows])

    zeros16 = jnp.zeros((LANES,), f32)

    zoffs = [k * LANES for k in range(D // LANES)]
    if D % LANES:
      zoffs.append(D - LANES)

    def zero_rows(j, _):
      for off in zoffs:
        zbuf[j, pl.ds(off, LANES)] = zeros16
      return 0
    lax.fori_loop(0, ZROWS, zero_rows, 0)
    for z in range(ROWS_PER_TILE // ZROWS):
      pltpu.sync_copy(
          zbuf, acc.at[pl.ds(sid * ROWS_PER_TILE + z * ZROWS, ZROWS)])

    def start_load(g, slot):
      base = g * NBUF
      pltpu.async_copy(src3.at[wid, pl.ds(base, NBUF)], es.at[slot],
                       lsem.at[slot])
      pltpu.async_copy(dst3.at[wid, pl.ds(base, NBUF)], ed.at[slot],
                       lsem.at[slot])

    def wait_load(slot):
      pltpu.make_async_copy(src3.at[wid, pl.ds(0, NBUF)], es.at[slot],
                            lsem.at[slot]).wait()
      pltpu.make_async_copy(dst3.at[wid, pl.ds(0, NBUF)], ed.at[slot],
                            lsem.at[slot]).wait()

    def start_gather(idxref, slot):
      pltpu.async_copy(tab.at[idxref], gbuf.at[slot], gsem.at[slot])

    def wait_gather(slot):
      pltpu.make_async_copy(
          tab.at[es.at[0, 0]], gbuf.at[slot], gsem.at[slot]).wait()

    def wait_scatter(slot):
      pltpu.make_async_copy(
          gbuf.at[slot], acc.at[ed.at[0, 0]], ssem.at[slot]).wait()

    start_load(0, 0)
    wait_load(0)
    start_load(1, 1)
    plsc.subcore_barrier()
    for b in range(LA):
      start_gather(es.at[0, b], b)

    def group_body(g, _):
      cur = lax.rem(g, 3)
      nxt = lax.rem(g + 1, 3)

      @pl.when(g + 1 < NG)
      def _():
        wait_load(nxt)

      for b in range(NBUF):
        j = g * NBUF + b
        jn = j + LA
        slot_n = (b + LA) % NBUF

        @pl.when(jn < nchunk)
        def _():
          @pl.when(j >= LA)
          def _():
            wait_scatter(slot_n)
          if b < NBUF - LA:
            start_gather(es.at[cur, b + LA], slot_n)
          else:
            start_gather(es.at[nxt, b + LA - NBUF], slot_n)

        wait_gather(b)
        pltpu.async_copy(gbuf.at[b], acc.at[ed.at[cur, b]], ssem.at[b],
                         add=True)

      @pl.when(g + 2 < NG)
      def _():
        start_load(g + 2, lax.rem(g + 2, 3))
      return 0
    lax.fori_loop(0, NG, group_body, 0)

    for b in range(NBUF):
      wait_scatter(b)

    plsc.subcore_barrier()

    rows = pl.ds(sid * ROWS_PER_TILE, ROWS_PER_TILE)
    ocol = pl.ds(cid * D, D)
    pltpu.sync_copy(acc.at[rows], pout.at[rows, ocol])

  return agg


_sc_agg64 = _make_sc_agg(64, split32=False, NBUF=4, LA=2)
_sc_agg48 = _make_sc_agg(40, split32=True, NBUF=8, LA=4)


def _norm(d):
  return lax.rsqrt(jnp.where(d > 0.0, d, 1.0))


def _prescale_body(nf, do_c, di_c, xs, nsb, ndb):
  ns = _norm(do_c[...])
  nd = _norm(di_c[...])
  x = nf[...]
  xs[...] = x * ns
  nsb[...] = jnp.broadcast_to(ns.astype(jnp.bfloat16), x.shape)
  ndb[...] = jnp.broadcast_to(nd.astype(jnp.bfloat16), x.shape)


def _layer_body(a1, ndb, nsb, w1, b1, w2, out):
  nd = ndb[...].astype(f32)
  h = jnp.dot(a1[...] * nd, w1[...], preferred_element_type=f32)
  h = jax.nn.relu(h + b1[...])
  ns = nsb[...].astype(f32)
  out[...] = jnp.dot(h, w2[...], preferred_element_type=f32) * ns


def _final_body(pp, ndb, b2, out):
  p = pp[...]
  nd = ndb[...].astype(f32)
  out[...] = (p[:, :C] + p[:, C:2 * C]) * nd[:, :C] + b2[...]


def _row_spec(width):
  return pl.BlockSpec((R_BLK, width), lambda r: (r, 0))


def _full_spec(shape):
  return pl.BlockSpec(shape, lambda r: tuple(0 for _ in shape))


_tc_prescale = pl.pallas_call(
    _prescale_body,
    grid=(GRID_R,),
    in_specs=[_row_spec(128), _row_spec(1), _row_spec(1)],
    out_specs=[_row_spec(128)] * 3,
    out_shape=[jax.ShapeDtypeStruct((N_T, 128), f32),
               jax.ShapeDtypeStruct((N_T, 128), jnp.bfloat16),
               jax.ShapeDtypeStruct((N_T, 128), jnp.bfloat16)],
)

_tc_layer = pl.pallas_call(
    _layer_body,
    grid=(GRID_R,),
    in_specs=[_row_spec(128), _row_spec(128), _row_spec(128),
              _full_spec((128, 128)), _full_spec((1, 128)),
              _full_spec((128, 128))],
    out_specs=[_row_spec(128)],
    out_shape=[jax.ShapeDtypeStruct((N_T, 128), f32)],
)

F_BLK = 2000
_tc_final = pl.pallas_call(
    _final_body,
    grid=(N // F_BLK,),
    in_specs=[pl.BlockSpec((F_BLK, 128), lambda r: (r, 0)),
              pl.BlockSpec((F_BLK, 128), lambda r: (r, 0)),
              _full_spec((1, C))],
    out_specs=[pl.BlockSpec((F_BLK, C), lambda r: (r, 0))],
    out_shape=[jax.ShapeDtypeStruct((N, C), f32)],
)


def kernel(n_feats, edge_index, W1, b1, W2, b2):
  do0, do1, di0, di1, srcp, dstp = _sc_degrees(edge_index)
  src16 = srcp.reshape(16, E_PAD // (16 * CHUNK), CHUNK)
  dst16 = dstp.reshape(16, E_PAD // (16 * CHUNK), CHUNK)
  src32 = srcp.reshape(32, E_PAD // (32 * CHUNK), CHUNK)
  dst32 = dstp.reshape(32, E_PAD // (32 * CHUNK), CHUNK)
  do_c = (do0 + do1).reshape(N_D, 1)
  di_c = (di0 + di1).reshape(N_D, 1)

  xs, nsb, ndb = _tc_prescale(n_feats, do_c, di_c)
  a1 = _sc_agg64(xs, src16, dst16)

  w2p = jnp.pad(W2, ((0, 0), (0, 128 - C)))
  (t2,) = _tc_layer(a1, ndb, nsb, W1, b1.reshape(1, H), w2p)

  pp = _sc_agg48(t2, src32, dst32)
  (outp,) = _tc_final(pp, ndb, b2.reshape(1, C))
  return outp
